# baseline (speedup 1.0000x reference)
import jax
import jax.numpy as jnp
from jax.experimental import pallas as pl
from jax.experimental.pallas import tpu as pltpu

_MASK = -10000.0
_HIGHLIGHT = 1.0
_BS = 512


def _body(lens_ref, x_ref, o_ref):
    b = pl.program_id(0)
    j = pl.program_id(1)
    l = lens_ref[b]
    h = x_ref.shape[2]
    rows = jax.lax.broadcasted_iota(jnp.int32, (_BS, h), 0) + j * _BS
    cols = jax.lax.broadcasted_iota(jnp.int32, (_BS, h), 1)
    maskrow = jnp.where(cols == h - 1, _HIGHLIGHT, _MASK).astype(jnp.float32)
    pad = rows >= l
    o_ref[0] = jnp.where(pad, maskrow, x_ref[0])


def kernel(x, lens):
    B, S, H = x.shape
    lens32 = lens.astype(jnp.int32)
    return pl.pallas_call(
        _body,
        grid=(B, S // _BS),
        in_specs=[
            pl.BlockSpec(memory_space=pltpu.SMEM),
            pl.BlockSpec((1, _BS, H), lambda b, j: (b, j, 0)),
        ],
        out_specs=pl.BlockSpec((1, _BS, H), lambda b, j: (b, j, 0)),
        out_shape=jax.ShapeDtypeStruct((B, S, H), jnp.float32),
    )(lens32, x)


# speedup vs baseline: 1.0932x; 1.0932x over previous
import jax
import jax.numpy as jnp
from jax import lax
from jax.experimental import pallas as pl
from jax.experimental.pallas import tpu as pltpu
from jax.experimental.pallas import tpu_sc as plsc

_MASK = -10000.0
_HIGHLIGHT = 1.0

_NC = 2
_NS = 16
_NW = _NC * _NS
_C = 32
_F = 32


def _sc_body(x_hbm, lens_hbm, fill_hbm, out_hbm,
             lens_v, fill_v, btile, buf0, buf1,
             lsem, fsem, g0, g1, s0, s1):
    rows_per_w = x_hbm.shape[0] // _NW
    wid = lax.axis_index("s") * _NC + lax.axis_index("c")
    base = pl.multiple_of(wid * rows_per_w, 8)
    r0 = (wid % 2) * rows_per_w

    col = pl.multiple_of(wid * 128, 128)
    pltpu.async_copy(
        lens_hbm.at[pl.ds(0, 8), pl.ds(col, 128)], lens_v, lsem
    ).wait()
    pltpu.async_copy(fill_hbm, fill_v, fsem).wait()

    l = lens_v[0, pl.ds(0, 16)][0]
    c = jnp.clip(l - r0, 0, rows_per_w)
    cr = c % 8
    cal = pl.multiple_of(c - cr, 8)
    has_mid = cr != 0
    cau = pl.multiple_of(cal + jnp.where(has_mid, 8, 0), 8)

    n_fill = rows_per_w - cau
    n_fchunks = n_fill // _F

    def fire_fill(g, _):
        start = pl.multiple_of(base + cau + g * _F, 8)
        pltpu.make_async_copy(fill_v, out_hbm.at[pl.ds(start, _F)], fsem).start()
        return 0

    lax.fori_loop(0, n_fchunks, fire_fill, 0)

    ftail = n_fill % _F
    fcur = base + cau + n_fchunks * _F
    for sz in (16, 8):
        cond = (ftail & sz) != 0

        @pl.when(cond)
        def _():
            start = pl.multiple_of(fcur, 8)
            pltpu.make_async_copy(
                fill_v.at[pl.ds(0, sz)], out_hbm.at[pl.ds(start, sz)], fsem
            ).start()

        fcur = fcur + jnp.where(cond, sz, 0)

    @pl.when(has_mid)
    def _():
        start = pl.multiple_of(base + cal, 8)
        pltpu.async_copy(x_hbm.at[pl.ds(start, 8)], btile, g0).wait()
        lanes = lax.broadcasted_iota(jnp.int32, (16,), 0)
        neg = jnp.full((16,), _MASK, jnp.float32)
        last = jnp.where(lanes == 15, _HIGHLIGHT, _MASK).astype(jnp.float32)
        h = x_hbm.shape[1]
        nch = h // 16
        for i in range(1, 8):
            @pl.when(i >= cr)
            def _():
                for j in range(nch):
                    btile[i, pl.ds(j * 16, 16)] = last if j == nch - 1 else neg
        pltpu.async_copy(btile, out_hbm.at[pl.ds(start, 8)], s0).wait()

    n_cchunks = cal // _C
    bufs = (buf0, buf1)
    gsems = (g0, g1)
    ssems = (s0, s1)

    @pl.when(n_cchunks > 0)
    def _():
        pltpu.make_async_copy(x_hbm.at[pl.ds(base, _C)], buf0, g0).start()

    @pl.when(n_cchunks > 1)
    def _():
        start = pl.multiple_of(base + _C, 8)
        pltpu.make_async_copy(x_hbm.at[pl.ds(start, _C)], buf1, g1).start()

    def pair_body(p, _):
        for j in range(2):
            k = p * 2 + j
            buf, gs, ss = bufs[j], gsems[j], ssems[j]

            @pl.when(k < n_cchunks)
            def _():
                start = pl.multiple_of(base + k * _C, 8)
                pltpu.make_async_copy(x_hbm.at[pl.ds(start, _C)], buf, gs).wait()
                desc = pltpu.async_copy(buf, out_hbm.at[pl.ds(start, _C)], ss)
                desc.wait()

            @pl.when(k + 2 < n_cchunks)
            def _():
                nstart = pl.multiple_of(base + (k + 2) * _C, 8)
                pltpu.make_async_copy(x_hbm.at[pl.ds(nstart, _C)], buf, gs).start()

        return 0

    n_pairs = (n_cchunks + 1) // 2
    lax.fori_loop(0, n_pairs, pair_body, 0)

    ctail = cal % _C
    ccur = base + n_cchunks * _C
    for sz in (16, 8):
        cond = (ctail & sz) != 0

        @pl.when(cond)
        def _():
            start = pl.multiple_of(ccur, 8)
            pltpu.async_copy(
                x_hbm.at[pl.ds(start, sz)], buf0.at[pl.ds(0, sz)], g0
            ).wait()
            pltpu.async_copy(
                buf0.at[pl.ds(0, sz)], out_hbm.at[pl.ds(start, sz)], s0
            ).wait()

        ccur = ccur + jnp.where(cond, sz, 0)

    def drain_fill(g, _):
        pltpu.make_async_copy(fill_v, out_hbm.at[pl.ds(base, _F)], fsem).wait()
        return 0

    lax.fori_loop(0, n_fchunks, drain_fill, 0)
    for sz in (16, 8):
        @pl.when((ftail & sz) != 0)
        def _():
            pltpu.make_async_copy(
                fill_v.at[pl.ds(0, sz)], out_hbm.at[pl.ds(base, sz)], fsem
            ).wait()


def kernel(x, lens):
    B, S, H = x.shape
    lens32 = lens.astype(jnp.int32)
    x2d = x.reshape(B * S, H)
    fill = jnp.full((_F, H), _MASK, dtype=jnp.float32).at[:, H - 1].set(_HIGHLIGHT)
    lens_pad = jnp.broadcast_to(
        jnp.repeat(lens32, 2 * 128)[None, :], (8, _NW * 128)
    )

    mesh = plsc.VectorSubcoreMesh(
        core_axis_name="c", subcore_axis_name="s", num_cores=_NC, num_subcores=_NS
    )
    out2d = pl.kernel(
        _sc_body,
        out_type=jax.ShapeDtypeStruct((B * S, H), jnp.float32),
        mesh=mesh,
        scratch_types=[
            pltpu.VMEM((8, 128), jnp.int32),
            pltpu.VMEM((_F, H), jnp.float32),
            pltpu.VMEM((8, H), jnp.float32),
            pltpu.VMEM((_C, H), jnp.float32),
            pltpu.VMEM((_C, H), jnp.float32),
            pltpu.SemaphoreType.DMA,
            pltpu.SemaphoreType.DMA,
            pltpu.SemaphoreType.DMA,
            pltpu.SemaphoreType.DMA,
            pltpu.SemaphoreType.DMA,
            pltpu.SemaphoreType.DMA,
        ],
    )(x2d, lens_pad, fill)
    return out2d.reshape(B, S, H)


# speedup vs baseline: 1.0938x; 1.0005x over previous
import jax
import jax.numpy as jnp
from jax import lax
from jax.experimental import pallas as pl
from jax.experimental.pallas import tpu as pltpu
from jax.experimental.pallas import tpu_sc as plsc

_MASK = -10000.0
_HIGHLIGHT = 1.0

_NC = 2
_NS = 16
_NW = _NC * _NS
_C = 48
_F = 16


def _sc_body(x_hbm, lens_hbm, fill_hbm, out_hbm,
             lens_v, fill_v, btile, buf0, buf1,
             lsem, fsem, g0, g1, s0, s1):
    rows_per_w = x_hbm.shape[0] // _NW
    wid = lax.axis_index("s") * _NC + lax.axis_index("c")
    base = pl.multiple_of(wid * rows_per_w, 8)
    r0 = (wid % 2) * rows_per_w

    col = pl.multiple_of(wid * 128, 128)
    pltpu.async_copy(
        lens_hbm.at[pl.ds(0, 8), pl.ds(col, 128)], lens_v, lsem
    ).wait()
    pltpu.async_copy(fill_hbm, fill_v, fsem).wait()

    l = lens_v[0, pl.ds(0, 16)][0]
    c = jnp.clip(l - r0, 0, rows_per_w)
    cr = c % 8
    cal = pl.multiple_of(c - cr, 8)
    has_mid = cr != 0
    cau = pl.multiple_of(cal + jnp.where(has_mid, 8, 0), 8)

    n_fill = rows_per_w - cau
    n_fchunks = n_fill // _F

    def fire_fill(g, _):
        start = pl.multiple_of(base + cau + g * _F, 8)
        pltpu.make_async_copy(fill_v, out_hbm.at[pl.ds(start, _F)], fsem).start()
        return 0

    lax.fori_loop(0, n_fchunks, fire_fill, 0)

    ftail = n_fill % _F
    fcur = base + cau + n_fchunks * _F
    for sz in (8,):
        cond = (ftail & sz) != 0

        @pl.when(cond)
        def _():
            start = pl.multiple_of(fcur, 8)
            pltpu.make_async_copy(
                fill_v.at[pl.ds(0, sz)], out_hbm.at[pl.ds(start, sz)], fsem
            ).start()

        fcur = fcur + jnp.where(cond, sz, 0)

    @pl.when(has_mid)
    def _():
        start = pl.multiple_of(base + cal, 8)
        pltpu.async_copy(x_hbm.at[pl.ds(start, 8)], btile, g0).wait()
        lanes = lax.broadcasted_iota(jnp.int32, (16,), 0)
        neg = jnp.full((16,), _MASK, jnp.float32)
        last = jnp.where(lanes == 15, _HIGHLIGHT, _MASK).astype(jnp.float32)
        h = x_hbm.shape[1]
        nch = h // 16
        for i in range(1, 8):
            @pl.when(i >= cr)
            def _():
                for j in range(nch):
                    btile[i, pl.ds(j * 16, 16)] = last if j == nch - 1 else neg
        pltpu.async_copy(btile, out_hbm.at[pl.ds(start, 8)], s0).wait()

    n_cchunks = cal // _C
    bufs = (buf0, buf1)
    gsems = (g0, g1)
    ssems = (s0, s1)

    @pl.when(n_cchunks > 0)
    def _():
        pltpu.make_async_copy(x_hbm.at[pl.ds(base, _C)], buf0, g0).start()

    @pl.when(n_cchunks > 1)
    def _():
        start = pl.multiple_of(base + _C, 8)
        pltpu.make_async_copy(x_hbm.at[pl.ds(start, _C)], buf1, g1).start()

    def pair_body(p, _):
        for j in range(2):
            k = p * 2 + j
            buf, gs, ss = bufs[j], gsems[j], ssems[j]

            @pl.when(k < n_cchunks)
            def _():
                start = pl.multiple_of(base + k * _C, 8)
                pltpu.make_async_copy(x_hbm.at[pl.ds(start, _C)], buf, gs).wait()
                pltpu.make_async_copy(buf, out_hbm.at[pl.ds(start, _C)], ss).start()

            @pl.when(k + 2 < n_cchunks)
            def _():
                pltpu.make_async_copy(buf, out_hbm.at[pl.ds(base, _C)], ss).wait()
                nstart = pl.multiple_of(base + (k + 2) * _C, 8)
                pltpu.make_async_copy(x_hbm.at[pl.ds(nstart, _C)], buf, gs).start()

        return 0

    n_pairs = (n_cchunks + 1) // 2
    lax.fori_loop(0, n_pairs, pair_body, 0)

    @pl.when(n_cchunks > 0)
    def _():
        pltpu.make_async_copy(buf0, out_hbm.at[pl.ds(base, _C)], s0).wait()

    @pl.when(n_cchunks > 1)
    def _():
        pltpu.make_async_copy(buf1, out_hbm.at[pl.ds(base, _C)], s1).wait()

    ctail = cal % _C
    ccur = base + n_cchunks * _C
    for sz in (32, 16, 8):
        cond = (ctail & sz) != 0

        @pl.when(cond)
        def _():
            start = pl.multiple_of(ccur, 8)
            pltpu.async_copy(
                x_hbm.at[pl.ds(start, sz)], buf0.at[pl.ds(0, sz)], g0
            ).wait()
            pltpu.async_copy(
                buf0.at[pl.ds(0, sz)], out_hbm.at[pl.ds(start, sz)], s0
            ).wait()

        ccur = ccur + jnp.where(cond, sz, 0)

    def drain_fill(g, _):
        pltpu.make_async_copy(fill_v, out_hbm.at[pl.ds(base, _F)], fsem).wait()
        return 0

    lax.fori_loop(0, n_fchunks, drain_fill, 0)
    for sz in (8,):
        @pl.when((ftail & sz) != 0)
        def _():
            pltpu.make_async_copy(
                fill_v.at[pl.ds(0, sz)], out_hbm.at[pl.ds(base, sz)], fsem
            ).wait()


def kernel(x, lens):
    B, S, H = x.shape
    lens32 = lens.astype(jnp.int32)
    x2d = x.reshape(B * S, H)
    fill = jnp.full((_F, H), _MASK, dtype=jnp.float32).at[:, H - 1].set(_HIGHLIGHT)
    lens_pad = jnp.broadcast_to(
        jnp.repeat(lens32, 2 * 128)[None, :], (8, _NW * 128)
    )

    mesh = plsc.VectorSubcoreMesh(
        core_axis_name="c", subcore_axis_name="s", num_cores=_NC, num_subcores=_NS
    )
    out2d = pl.kernel(
        _sc_body,
        out_type=jax.ShapeDtypeStruct((B * S, H), jnp.float32),
        mesh=mesh,
        scratch_types=[
            pltpu.VMEM((8, 128), jnp.int32),
            pltpu.VMEM((_F, H), jnp.float32),
            pltpu.VMEM((8, H), jnp.float32),
            pltpu.VMEM((_C, H), jnp.float32),
            pltpu.VMEM((_C, H), jnp.float32),
            pltpu.SemaphoreType.DMA,
            pltpu.SemaphoreType.DMA,
            pltpu.SemaphoreType.DMA,
            pltpu.SemaphoreType.DMA,
            pltpu.SemaphoreType.DMA,
            pltpu.SemaphoreType.DMA,
        ],
    )(x2d, lens_pad, fill)
    return out2d.reshape(B, S, H)


# speedup vs baseline: 1.1056x; 1.0108x over previous
import jax
import jax.numpy as jnp
from jax import lax
from jax.experimental import pallas as pl
from jax.experimental.pallas import tpu as pltpu
from jax.experimental.pallas import tpu_sc as plsc

_MASK = -10000.0
_HIGHLIGHT = 1.0

_NC = 2
_NS = 16
_NW = _NC * _NS
_C = 48
_F = 16


def _sc_body(x_hbm, lens_hbm, fill_hbm, out_hbm,
             lens_v, fill_v, btile, buf0, buf1,
             lsem, fsem, g0, g1, s0, s1):
    rows_per_w = x_hbm.shape[0] // _NW
    wid = lax.axis_index("s") * _NC + lax.axis_index("c")
    b = wid // 2
    half = (wid % 2) ^ (b % 2)
    r0 = half * rows_per_w
    base = pl.multiple_of(b * (2 * rows_per_w) + r0, 8)

    col = pl.multiple_of(wid * 128, 128)
    dl = pltpu.make_async_copy(
        lens_hbm.at[pl.ds(0, 8), pl.ds(col, 128)], lens_v, lsem
    )
    dl.start()
    df = pltpu.make_async_copy(fill_hbm, fill_v, fsem)
    df.start()
    dl.wait()
    df.wait()

    l = lens_v[0, pl.ds(0, 16)][0]
    c = jnp.clip(l - r0, 0, rows_per_w)
    cr = c % 8
    cal = pl.multiple_of(c - cr, 8)
    has_mid = cr != 0
    cau = pl.multiple_of(cal + jnp.where(has_mid, 8, 0), 8)

    n_fill = rows_per_w - cau
    n_fchunks = n_fill // _F

    def fire_fill(g, _):
        start = pl.multiple_of(base + cau + g * _F, 8)
        pltpu.make_async_copy(fill_v, out_hbm.at[pl.ds(start, _F)], fsem).start()
        return 0

    lax.fori_loop(0, n_fchunks, fire_fill, 0)

    ftail = n_fill % _F
    fcur = base + cau + n_fchunks * _F
    for sz in (8,):
        cond = (ftail & sz) != 0

        @pl.when(cond)
        def _():
            start = pl.multiple_of(fcur, 8)
            pltpu.make_async_copy(
                fill_v.at[pl.ds(0, sz)], out_hbm.at[pl.ds(start, sz)], fsem
            ).start()

        fcur = fcur + jnp.where(cond, sz, 0)

    @pl.when(has_mid)
    def _():
        start = pl.multiple_of(base + cal, 8)
        pltpu.async_copy(x_hbm.at[pl.ds(start, 8)], btile, g0).wait()
        lanes = lax.broadcasted_iota(jnp.int32, (16,), 0)
        neg = jnp.full((16,), _MASK, jnp.float32)
        last = jnp.where(lanes == 15, _HIGHLIGHT, _MASK).astype(jnp.float32)
        h = x_hbm.shape[1]
        nch = h // 16
        for i in range(1, 8):
            @pl.when(i >= cr)
            def _():
                for j in range(nch):
                    btile[i, pl.ds(j * 16, 16)] = last if j == nch - 1 else neg
        pltpu.async_copy(btile, out_hbm.at[pl.ds(start, 8)], s0).wait()

    n_cchunks = cal // _C
    bufs = (buf0, buf1)
    gsems = (g0, g1)
    ssems = (s0, s1)

    @pl.when(n_cchunks > 0)
    def _():
        pltpu.make_async_copy(x_hbm.at[pl.ds(base, _C)], buf0, g0).start()

    @pl.when(n_cchunks > 1)
    def _():
        start = pl.multiple_of(base + _C, 8)
        pltpu.make_async_copy(x_hbm.at[pl.ds(start, _C)], buf1, g1).start()

    def pair_body(p, _):
        for j in range(2):
            k = p * 2 + j
            buf, gs, ss = bufs[j], gsems[j], ssems[j]

            @pl.when(k < n_cchunks)
            def _():
                start = pl.multiple_of(base + k * _C, 8)
                pltpu.make_async_copy(x_hbm.at[pl.ds(start, _C)], buf, gs).wait()
                pltpu.make_async_copy(buf, out_hbm.at[pl.ds(start, _C)], ss).start()

            @pl.when(k + 2 < n_cchunks)
            def _():
                pltpu.make_async_copy(buf, out_hbm.at[pl.ds(base, _C)], ss).wait()
                nstart = pl.multiple_of(base + (k + 2) * _C, 8)
                pltpu.make_async_copy(x_hbm.at[pl.ds(nstart, _C)], buf, gs).start()

        return 0

    n_pairs = (n_cchunks + 1) // 2
    lax.fori_loop(0, n_pairs, pair_body, 0)

    @pl.when(n_cchunks > 0)
    def _():
        pltpu.make_async_copy(buf0, out_hbm.at[pl.ds(base, _C)], s0).wait()

    @pl.when(n_cchunks > 1)
    def _():
        pltpu.make_async_copy(buf1, out_hbm.at[pl.ds(base, _C)], s1).wait()

    ctail = cal % _C
    ccur = base + n_cchunks * _C
    for sz in (32, 16, 8):
        cond = (ctail & sz) != 0

        @pl.when(cond)
        def _():
            start = pl.multiple_of(ccur, 8)
            pltpu.async_copy(
                x_hbm.at[pl.ds(start, sz)], buf0.at[pl.ds(0, sz)], g0
            ).wait()
            pltpu.async_copy(
                buf0.at[pl.ds(0, sz)], out_hbm.at[pl.ds(start, sz)], s0
            ).wait()

        ccur = ccur + jnp.where(cond, sz, 0)

    def drain_fill(g, _):
        pltpu.make_async_copy(fill_v, out_hbm.at[pl.ds(base, _F)], fsem).wait()
        return 0

    lax.fori_loop(0, n_fchunks, drain_fill, 0)
    for sz in (8,):
        @pl.when((ftail & sz) != 0)
        def _():
            pltpu.make_async_copy(
                fill_v.at[pl.ds(0, sz)], out_hbm.at[pl.ds(base, sz)], fsem
            ).wait()


def kernel(x, lens):
    B, S, H = x.shape
    lens32 = lens.astype(jnp.int32)
    x2d = x.reshape(B * S, H)
    fill = jnp.full((_F, H), _MASK, dtype=jnp.float32).at[:, H - 1].set(_HIGHLIGHT)
    lens_pad = jnp.broadcast_to(
        jnp.repeat(lens32, 2 * 128)[None, :], (8, _NW * 128)
    )

    mesh = plsc.VectorSubcoreMesh(
        core_axis_name="c", subcore_axis_name="s", num_cores=_NC, num_subcores=_NS
    )
    out2d = pl.kernel(
        _sc_body,
        out_type=jax.ShapeDtypeStruct((B * S, H), jnp.float32),
        mesh=mesh,
        scratch_types=[
            pltpu.VMEM((8, 128), jnp.int32),
            pltpu.VMEM((_F, H), jnp.float32),
            pltpu.VMEM((8, H), jnp.float32),
            pltpu.VMEM((_C, H), jnp.float32),
            pltpu.VMEM((_C, H), jnp.float32),
            pltpu.SemaphoreType.DMA,
            pltpu.SemaphoreType.DMA,
            pltpu.SemaphoreType.DMA,
            pltpu.SemaphoreType.DMA,
            pltpu.SemaphoreType.DMA,
            pltpu.SemaphoreType.DMA,
        ],
    )(x2d, lens_pad, fill)
    return out2d.reshape(B, S, H)
